# baseline (speedup 1.0000x reference)
import functools

import jax
import jax.numpy as jnp
import numpy as np
from jax import lax
from jax.experimental import pallas as pl
from jax.experimental.pallas import tpu as pltpu
from jax.experimental.pallas import tpu_sc as plsc

_INFO = plsc.get_sparse_core_info()
_NC = _INFO.num_cores
_NS = _INFO.num_subcores
_L = _INFO.num_lanes
_NW = _NC * _NS

_K = 128
_NB = 4


def _emb_body(nseq, d, scale, idx_hbm, table_hbm, out_hbm, idx_v, in_bufs,
              out_bufs, gsems, osems):
    wid = lax.axis_index("s") * _NC + lax.axis_index("c")
    nfb = d // 8
    biota = lax.iota(jnp.int32, _L)
    bidxs = [biota + bg * _L for bg in range(_K // _L)]

    def stage(st, carry):
        pltpu.sync_copy(idx_hbm.at[st, wid], idx_v.at[st])
        return carry

    lax.fori_loop(0, nseq // 8, stage, 0)

    def start_gather(j, b):
        pltpu.make_async_copy(
            table_hbm.at[idx_v.at[j // 8, j % 8]], in_bufs[b], gsems[b]).start()

    def wait_gather(j, b):
        pltpu.make_async_copy(
            table_hbm.at[idx_v.at[j // 8, j % 8]], in_bufs[b], gsems[b]).wait()

    def out_copies(j, b, fn):
        def per_fb(fb, carry):
            getattr(pltpu.make_async_copy(
                out_bufs[b].at[fb], out_hbm.at[j, fb, wid], osems[b]), fn)()
            return carry

        lax.fori_loop(0, nfb, per_fb, 0)

    def transpose_scale(b):
        src = in_bufs[b]
        dst = out_bufs[b]

        def col_body(t, carry):
            col = jnp.full((_L,), t, jnp.int32)
            for bg in range(_K // _L):
                vals = plsc.load_gather(src, [bidxs[bg], col])
                dst[t // 8, t % 8, pl.ds(bg * _L, _L)] = vals * scale
            return carry

        lax.fori_loop(0, d, col_body, 0)

    ngroup = nseq // _NB

    for b in range(_NB):
        start_gather(b, b)

    def group(g, carry):
        for b in range(_NB):
            j = g * _NB + b
            wait_gather(j, b)

            @pl.when(g > 0)
            def _():
                out_copies(j - _NB, b, "wait")

            transpose_scale(b)
            out_copies(j, b, "start")

            @pl.when(j + _NB < nseq)
            def _():
                start_gather(j + _NB, b)

        return carry

    lax.fori_loop(0, ngroup, group, 0)

    for b in range(_NB):
        out_copies(nseq - _NB + b, b, "wait")


def kernel(input, table):
    bt, s = input.shape
    v, d = table.shape
    assert bt % (_NW * _K // _NW) == 0 and d % _L == 0 and d % 8 == 0
    nbb = bt // _K
    assert nbb == _NW and s % 8 == 0
    scale = np.float32(np.sqrt(d))

    idx4 = (input.astype(jnp.int32)
            .reshape(nbb, _K, s // 8, 8).transpose(2, 0, 3, 1))

    mesh = plsc.VectorSubcoreMesh(core_axis_name="c", subcore_axis_name="s")

    def body(idx_hbm, table_hbm, out_hbm, idx_v, *rest):
        in_bufs = rest[:_NB]
        out_bufs = rest[_NB:2 * _NB]
        gsems = rest[2 * _NB:3 * _NB]
        osems = rest[3 * _NB:]
        _emb_body(s, d, scale, idx_hbm, table_hbm, out_hbm, idx_v, in_bufs,
                  out_bufs, gsems, osems)

    run = pl.kernel(
        body,
        mesh=mesh,
        out_type=jax.ShapeDtypeStruct((s, d // 8, nbb, 8, _K), jnp.float32),
        scratch_types=(
            [pltpu.VMEM((s // 8, 8, _K), jnp.int32)]
            + [pltpu.VMEM((_K, d), jnp.float32) for _ in range(_NB)]
            + [pltpu.VMEM((d // 8, 8, _K), jnp.float32) for _ in range(_NB)]
            + [pltpu.SemaphoreType.DMA] * (2 * _NB)
        ),
        compiler_params=pltpu.CompilerParams(
            use_tc_tiling_on_sc=False, needs_layout_passes=False),
    )
    out5 = run(idx4, table)
    return out5.transpose(2, 4, 0, 1, 3).reshape(bt, s, d)


# speedup vs baseline: 1.1217x; 1.1217x over previous
import functools

import jax
import jax.numpy as jnp
import numpy as np
from jax import lax
from jax.experimental import pallas as pl
from jax.experimental.pallas import tpu as pltpu
from jax.experimental.pallas import tpu_sc as plsc

_INFO = plsc.get_sparse_core_info()
_NC = _INFO.num_cores
_NS = _INFO.num_subcores
_L = _INFO.num_lanes
_NW = _NC * _NS

_K = 128
_NB = 4


def _emb_body(nseq, d, scale, idx_hbm, table_hbm, out_hbm, idx_v, in_bufs,
              out_bufs, gsems, osems):
    wid = lax.axis_index("s") * _NC + lax.axis_index("c")
    nfb = d // 8
    biota = lax.iota(jnp.int32, _L)
    qvs = [(biota + q * _L) * _K for q in range(d // _L)]

    def stage(st, carry):
        pltpu.sync_copy(idx_hbm.at[st, wid], idx_v.at[st])
        return carry

    lax.fori_loop(0, nseq // 8, stage, 0)

    def start_gather(j, b):
        pltpu.make_async_copy(
            table_hbm.at[idx_v.at[j // 8, j % 8]], in_bufs[b], gsems[b]).start()

    def wait_gather(j, b):
        pltpu.make_async_copy(
            table_hbm.at[idx_v.at[j // 8, j % 8]], in_bufs[b], gsems[b]).wait()

    def out_copies(j, b, fn):
        for fb in range(nfb):
            getattr(pltpu.make_async_copy(
                out_bufs[b].at[pl.ds(fb * 8 * _K, 8 * _K)],
                out_hbm.at[j, fb, wid], osems[b]), fn)()

    def transpose_scale(b):
        src = in_bufs[b]
        dst = out_bufs[b]
        nq = d // _L

        def row_body(r, carry):
            for rr in range(2):
                for q in range(nq):
                    row = r * 2 + rr
                    vals = src[row, pl.ds(q * _L, _L)] * scale
                    plsc.store_scatter(dst, [qvs[q] + row], vals)
            return carry

        lax.fori_loop(0, _K // 2, row_body, 0)

    ngroup = nseq // _NB

    for b in range(_NB):
        start_gather(b, b)

    def group(g, carry):
        for b in range(_NB):
            j = g * _NB + b
            wait_gather(j, b)

            @pl.when(g > 0)
            def _():
                out_copies(j - _NB, b, "wait")

            transpose_scale(b)
            out_copies(j, b, "start")

            @pl.when(j + _NB < nseq)
            def _():
                start_gather(j + _NB, b)

        return carry

    lax.fori_loop(0, ngroup, group, 0)

    for b in range(_NB):
        out_copies(nseq - _NB + b, b, "wait")


def kernel(input, table):
    bt, s = input.shape
    v, d = table.shape
    assert bt % (_NW * _K // _NW) == 0 and d % _L == 0 and d % 8 == 0
    nbb = bt // _K
    assert nbb == _NW and s % 8 == 0
    scale = np.float32(np.sqrt(d))

    idx4 = (input.astype(jnp.int32)
            .reshape(nbb, _K, s // 8, 8).transpose(2, 0, 3, 1))

    mesh = plsc.VectorSubcoreMesh(core_axis_name="c", subcore_axis_name="s")

    def body(idx_hbm, table_hbm, out_hbm, idx_v, *rest):
        in_bufs = rest[:_NB]
        out_bufs = rest[_NB:2 * _NB]
        gsems = rest[2 * _NB:3 * _NB]
        osems = rest[3 * _NB:]
        _emb_body(s, d, scale, idx_hbm, table_hbm, out_hbm, idx_v, in_bufs,
                  out_bufs, gsems, osems)

    run = pl.kernel(
        body,
        mesh=mesh,
        out_type=jax.ShapeDtypeStruct((s, d // 8, nbb, 8 * _K), jnp.float32),
        scratch_types=(
            [pltpu.VMEM((s // 8, 8, _K), jnp.int32)]
            + [pltpu.VMEM((_K, d), jnp.float32) for _ in range(_NB)]
            + [pltpu.VMEM((d * _K,), jnp.float32) for _ in range(_NB)]
            + [pltpu.SemaphoreType.DMA] * (2 * _NB)
        ),
        compiler_params=pltpu.CompilerParams(
            use_tc_tiling_on_sc=False, needs_layout_passes=False),
    )
    out5 = run(idx4, table).reshape(s, d // 8, nbb, 8, _K)
    return out5.transpose(2, 4, 0, 1, 3).reshape(bt, s, d)


# speedup vs baseline: 1.6809x; 1.4986x over previous
import functools

import jax
import jax.numpy as jnp
import numpy as np
from jax import lax
from jax.experimental import pallas as pl
from jax.experimental.pallas import tpu as pltpu
from jax.experimental.pallas import tpu_sc as plsc

_INFO = plsc.get_sparse_core_info()
_NC = _INFO.num_cores
_NS = _INFO.num_subcores
_L = _INFO.num_lanes
_NW = _NC * _NS

_K = 128
_NB = 4


def _emb_body(nseq, d, scale, idx_hbm, table_hbm, out_hbm, idx_v, in_bufs,
              out_bufs, gsems, osems):
    wid = lax.axis_index("s") * _NC + lax.axis_index("c")
    nfb = d // 8
    biota = lax.iota(jnp.int32, _L)
    cvecs = [(dg + biota) % _L for dg in range(_L)]
    dvecs = [((dg + biota) % _L) * _K + biota for dg in range(_L)]

    def stage(st, carry):
        pltpu.sync_copy(idx_hbm.at[st, wid], idx_v.at[st])
        return carry

    lax.fori_loop(0, nseq // 8, stage, 0)

    def start_gather(j, b):
        pltpu.make_async_copy(
            table_hbm.at[idx_v.at[j // 8, j % 8]], in_bufs[b], gsems[b]).start()

    def wait_gather(j, b):
        pltpu.make_async_copy(
            table_hbm.at[idx_v.at[j // 8, j % 8]], in_bufs[b], gsems[b]).wait()

    def out_copies(j, b, fn):
        for fb in range(nfb):
            getattr(pltpu.make_async_copy(
                out_bufs[b].at[pl.ds(fb * 8 * _K, 8 * _K)],
                out_hbm.at[j, fb, wid], osems[b]), fn)()

    def transpose_scale(b):
        src = in_bufs[b]
        dst = out_bufs[b]
        nq = d // _L

        def block_body(t, carry):
            rb = t // nq
            q = t % nq
            rvec = biota + rb * _L
            dbase = q * _L * _K + rb * _L
            for dg in range(_L):
                vals = plsc.load_gather(src, [rvec, cvecs[dg] + q * _L])
                plsc.store_scatter(dst, [dvecs[dg] + dbase], vals * scale)
            return carry

        lax.fori_loop(0, (_K // _L) * nq, block_body, 0)

    ngroup = nseq // _NB

    for b in range(_NB):
        start_gather(b, b)

    def group(g, carry):
        for b in range(_NB):
            j = g * _NB + b
            wait_gather(j, b)

            @pl.when(g > 0)
            def _():
                out_copies(j - _NB, b, "wait")

            transpose_scale(b)
            out_copies(j, b, "start")

            @pl.when(j + _NB < nseq)
            def _():
                start_gather(j + _NB, b)

        return carry

    lax.fori_loop(0, ngroup, group, 0)

    for b in range(_NB):
        out_copies(nseq - _NB + b, b, "wait")


def kernel(input, table):
    bt, s = input.shape
    v, d = table.shape
    assert bt % (_NW * _K // _NW) == 0 and d % _L == 0 and d % 8 == 0
    nbb = bt // _K
    assert nbb == _NW and s % 8 == 0
    scale = np.float32(np.sqrt(d))

    idx4 = (input.astype(jnp.int32)
            .reshape(nbb, _K, s // 8, 8).transpose(2, 0, 3, 1))

    mesh = plsc.VectorSubcoreMesh(core_axis_name="c", subcore_axis_name="s")

    def body(idx_hbm, table_hbm, out_hbm, idx_v, *rest):
        in_bufs = rest[:_NB]
        out_bufs = rest[_NB:2 * _NB]
        gsems = rest[2 * _NB:3 * _NB]
        osems = rest[3 * _NB:]
        _emb_body(s, d, scale, idx_hbm, table_hbm, out_hbm, idx_v, in_bufs,
                  out_bufs, gsems, osems)

    run = pl.kernel(
        body,
        mesh=mesh,
        out_type=jax.ShapeDtypeStruct((s, d // 8, nbb, 8 * _K), jnp.float32),
        scratch_types=(
            [pltpu.VMEM((s // 8, 8, _K), jnp.int32)]
            + [pltpu.VMEM((_K, d), jnp.float32) for _ in range(_NB)]
            + [pltpu.VMEM((d * _K,), jnp.float32) for _ in range(_NB)]
            + [pltpu.SemaphoreType.DMA] * (2 * _NB)
        ),
        compiler_params=pltpu.CompilerParams(
            use_tc_tiling_on_sc=False, needs_layout_passes=False),
    )
    out5 = run(idx4, table).reshape(s, d // 8, nbb, 8, _K)
    return out5.transpose(2, 4, 0, 1, 3).reshape(bt, s, d)


# speedup vs baseline: 2.5646x; 1.5257x over previous
import functools

import jax
import jax.numpy as jnp
import numpy as np
from jax import lax
from jax.experimental import pallas as pl
from jax.experimental.pallas import tpu as pltpu
from jax.experimental.pallas import tpu_sc as plsc

_INFO = plsc.get_sparse_core_info()
_NC = _INFO.num_cores
_NS = _INFO.num_subcores
_L = _INFO.num_lanes
_NW = _NC * _NS

_K = 128
_NB = 4


def _emb_body(nseq, d, scale, idx_hbm, table_hbm, out_hbm, idx_v, in_bufs,
              out_bufs, gsems, osems):
    wid = lax.axis_index("s") * _NC + lax.axis_index("c")
    nfb = d // 8
    biota = lax.iota(jnp.int32, _L)
    cvecs = [(dg + biota) % _L for dg in range(_L)]
    dvecs = [((dg + biota) % _L) * _K + biota for dg in range(_L)]

    def stage(st, carry):
        pltpu.sync_copy(idx_hbm.at[st, wid], idx_v.at[st])
        return carry

    lax.fori_loop(0, nseq // 8, stage, 0)

    def start_gather(j, b):
        pltpu.make_async_copy(
            table_hbm.at[idx_v.at[j // 8, j % 8]], in_bufs[b], gsems[b]).start()

    def wait_gather(j, b):
        pltpu.make_async_copy(
            table_hbm.at[idx_v.at[j // 8, j % 8]], in_bufs[b], gsems[b]).wait()

    def out_copies(j, b, fn):
        for fb in range(nfb):
            getattr(pltpu.make_async_copy(
                out_bufs[b].at[pl.ds(fb * 8 * _K, 8 * _K)],
                out_hbm.at[j, fb, wid], osems[b]), fn)()

    def transpose_scale(b):
        src = in_bufs[b]
        dst = out_bufs[b]
        nq = d // _L

        @plsc.parallel_loop(0, (_K // _L) * nq)
        def block_body(t):
            rb = t // nq
            q = t % nq
            rvec = biota + rb * _L
            dbase = q * _L * _K + rb * _L
            for dg in range(_L):
                vals = plsc.load_gather(src, [rvec, cvecs[dg] + q * _L])
                plsc.store_scatter(dst, [dvecs[dg] + dbase], vals * scale)

    ngroup = nseq // _NB

    for b in range(_NB):
        start_gather(b, b)

    def group(g, carry):
        for b in range(_NB):
            j = g * _NB + b
            wait_gather(j, b)

            @pl.when(g > 0)
            def _():
                out_copies(j - _NB, b, "wait")

            transpose_scale(b)
            out_copies(j, b, "start")

            @pl.when(j + _NB < nseq)
            def _():
                start_gather(j + _NB, b)

        return carry

    lax.fori_loop(0, ngroup, group, 0)

    for b in range(_NB):
        out_copies(nseq - _NB + b, b, "wait")


def kernel(input, table):
    bt, s = input.shape
    v, d = table.shape
    assert bt % (_NW * _K // _NW) == 0 and d % _L == 0 and d % 8 == 0
    nbb = bt // _K
    assert nbb == _NW and s % 8 == 0
    scale = np.float32(np.sqrt(d))

    idx4 = (input.astype(jnp.int32)
            .reshape(nbb, _K, s // 8, 8).transpose(2, 0, 3, 1))

    mesh = plsc.VectorSubcoreMesh(core_axis_name="c", subcore_axis_name="s")

    def body(idx_hbm, table_hbm, out_hbm, idx_v, *rest):
        in_bufs = rest[:_NB]
        out_bufs = rest[_NB:2 * _NB]
        gsems = rest[2 * _NB:3 * _NB]
        osems = rest[3 * _NB:]
        _emb_body(s, d, scale, idx_hbm, table_hbm, out_hbm, idx_v, in_bufs,
                  out_bufs, gsems, osems)

    run = pl.kernel(
        body,
        mesh=mesh,
        out_type=jax.ShapeDtypeStruct((s, d // 8, nbb, 8 * _K), jnp.float32),
        scratch_types=(
            [pltpu.VMEM((s // 8, 8, _K), jnp.int32)]
            + [pltpu.VMEM((_K, d), jnp.float32) for _ in range(_NB)]
            + [pltpu.VMEM((d * _K,), jnp.float32) for _ in range(_NB)]
            + [pltpu.SemaphoreType.DMA] * (2 * _NB)
        ),
        compiler_params=pltpu.CompilerParams(
            use_tc_tiling_on_sc=False, needs_layout_passes=False),
    )
    out5 = run(idx4, table).reshape(s, d // 8, nbb, 8, _K)
    return out5.transpose(2, 4, 0, 1, 3).reshape(bt, s, d)
